# baseline (speedup 1.0000x reference)
import functools

import jax
import jax.numpy as jnp
from jax import lax
from jax.experimental import pallas as pl
from jax.experimental.pallas import tpu as pltpu
from jax.experimental.pallas import tpu_sc as plsc

_DIM = 128
_BATCH = 4096
_NTILES = 16
_CHUNKS = 4
_B_PER_T = _BATCH // _NTILES
_B_PER_C = _B_PER_T // _CHUNKS


def _make_gather():
    mesh = plsc.VectorSubcoreMesh(
        core_axis_name="c", subcore_axis_name="s", num_cores=1
    )

    @functools.partial(
        pl.kernel,
        mesh=mesh,
        out_type=jax.ShapeDtypeStruct((_BATCH, _DIM), jnp.float32),
        scratch_types=[
            pltpu.VMEM((_CHUNKS, _B_PER_C), jnp.int32),
            pltpu.VMEM((2, _B_PER_C, _DIM), jnp.float32),
            pltpu.SemaphoreType.DMA,
            pltpu.SemaphoreType.DMA,
            pltpu.SemaphoreType.DMA,
            pltpu.SemaphoreType.DMA,
        ],
    )
    def gather_kernel(idx_hbm, table_hbm, out_hbm, idx_v, rows_v, g0, g1, s0, s1):
        tid = lax.axis_index("s")
        base = tid * _B_PER_T
        gsems = (g0, g1)
        ssems = (s0, s1)
        pltpu.sync_copy(idx_hbm.at[tid], idx_v)

        gathers = [
            pltpu.make_async_copy(
                table_hbm.at[idx_v.at[c]], rows_v.at[c % 2], gsems[c % 2]
            )
            for c in range(_CHUNKS)
        ]
        stores = [
            pltpu.make_async_copy(
                rows_v.at[c % 2],
                out_hbm.at[pl.ds(base + c * _B_PER_C, _B_PER_C)],
                ssems[c % 2],
            )
            for c in range(_CHUNKS)
        ]
        gathers[0].start()
        for c in range(_CHUNKS):
            gathers[c].wait()
            if c + 1 < _CHUNKS:
                if c + 1 >= 2:
                    stores[c - 1].wait()
                gathers[c + 1].start()
            stores[c].start()
        stores[_CHUNKS - 2].wait()
        stores[_CHUNKS - 1].wait()

    return gather_kernel


def kernel(times, emb_weight):
    gather = _make_gather()
    idx = times.astype(jnp.int32).reshape(_NTILES, _CHUNKS, _B_PER_C)
    return gather(idx, emb_weight)


# speedup vs baseline: 1.0557x; 1.0557x over previous
import functools

import jax
import jax.numpy as jnp
from jax import lax
from jax.experimental import pallas as pl
from jax.experimental.pallas import tpu as pltpu
from jax.experimental.pallas import tpu_sc as plsc

_DIM = 128
_BATCH = 4096
_NTILES = 16
_CHUNKS = 2
_B_PER_T = _BATCH // _NTILES
_B_PER_C = _B_PER_T // _CHUNKS


def _make_gather():
    mesh = plsc.VectorSubcoreMesh(
        core_axis_name="c", subcore_axis_name="s", num_cores=1
    )

    @functools.partial(
        pl.kernel,
        mesh=mesh,
        out_type=jax.ShapeDtypeStruct((_BATCH, _DIM), jnp.float32),
        scratch_types=[
            pltpu.VMEM((_CHUNKS, _B_PER_C), jnp.int32),
            pltpu.VMEM((2, _B_PER_C, _DIM), jnp.float32),
            pltpu.SemaphoreType.DMA,
            pltpu.SemaphoreType.DMA,
            pltpu.SemaphoreType.DMA,
            pltpu.SemaphoreType.DMA,
        ],
    )
    def gather_kernel(idx_hbm, table_hbm, out_hbm, idx_v, rows_v, g0, g1, s0, s1):
        tid = lax.axis_index("s")
        base = tid * _B_PER_T
        gsems = (g0, g1)
        ssems = (s0, s1)
        pltpu.sync_copy(idx_hbm.at[tid], idx_v)

        gathers = [
            pltpu.make_async_copy(
                table_hbm.at[idx_v.at[c]], rows_v.at[c % 2], gsems[c % 2]
            )
            for c in range(_CHUNKS)
        ]
        stores = [
            pltpu.make_async_copy(
                rows_v.at[c % 2],
                out_hbm.at[pl.ds(base + c * _B_PER_C, _B_PER_C)],
                ssems[c % 2],
            )
            for c in range(_CHUNKS)
        ]
        gathers[0].start()
        for c in range(_CHUNKS):
            gathers[c].wait()
            if c + 1 < _CHUNKS:
                if c + 1 >= 2:
                    stores[c - 1].wait()
                gathers[c + 1].start()
            stores[c].start()
        stores[_CHUNKS - 2].wait()
        stores[_CHUNKS - 1].wait()

    return gather_kernel


def kernel(times, emb_weight):
    gather = _make_gather()
    idx = times.astype(jnp.int32).reshape(_NTILES, _CHUNKS, _B_PER_C)
    return gather(idx, emb_weight)


# speedup vs baseline: 1.0818x; 1.0248x over previous
import functools

import jax
import jax.numpy as jnp
from jax import lax
from jax.experimental import pallas as pl
from jax.experimental.pallas import tpu as pltpu
from jax.experimental.pallas import tpu_sc as plsc

_DIM = 128
_BATCH = 4096
_NTILES = 16
_B_PER_T = _BATCH // _NTILES


def _make_gather():
    mesh = plsc.VectorSubcoreMesh(
        core_axis_name="c", subcore_axis_name="s", num_cores=1
    )

    @functools.partial(
        pl.kernel,
        mesh=mesh,
        out_type=jax.ShapeDtypeStruct((_BATCH, _DIM), jnp.float32),
        scratch_types=[
            pltpu.VMEM((_B_PER_T,), jnp.int32),
            pltpu.VMEM((_B_PER_T, _DIM), jnp.float32),
            pltpu.SemaphoreType.DMA,
        ],
    )
    def gather_kernel(idx_hbm, table_hbm, out_hbm, idx_v, rows_v, sem):
        tid = lax.axis_index("s")
        base = tid * _B_PER_T
        pltpu.sync_copy(idx_hbm.at[tid], idx_v)
        pltpu.async_copy(table_hbm.at[idx_v], rows_v, sem).wait()
        pltpu.sync_copy(rows_v, out_hbm.at[pl.ds(base, _B_PER_T)])

    return gather_kernel


def kernel(times, emb_weight):
    gather = _make_gather()
    idx = times.astype(jnp.int32).reshape(_NTILES, _B_PER_T)
    return gather(idx, emb_weight)


# speedup vs baseline: 1.0829x; 1.0010x over previous
import functools

import jax
import jax.numpy as jnp
from jax import lax
from jax.experimental import pallas as pl
from jax.experimental.pallas import tpu as pltpu
from jax.experimental.pallas import tpu_sc as plsc

_DIM = 128
_BATCH = 4096
_NTILES = 16
_B_PER_T = _BATCH // _NTILES


def _make_gather():
    mesh = plsc.VectorSubcoreMesh(
        core_axis_name="c", subcore_axis_name="s", num_cores=1
    )

    @functools.partial(
        pl.kernel,
        mesh=mesh,
        out_type=jax.ShapeDtypeStruct((_BATCH, _DIM), jnp.float32),
        compiler_params=pltpu.CompilerParams(
            disable_bounds_checks=True,
            disable_semaphore_checks=True,
        ),
        scratch_types=[
            pltpu.VMEM((_B_PER_T,), jnp.int32),
            pltpu.VMEM((_B_PER_T, _DIM), jnp.float32),
            pltpu.SemaphoreType.DMA,
        ],
    )
    def gather_kernel(idx_hbm, table_hbm, out_hbm, idx_v, rows_v, sem):
        tid = lax.axis_index("s")
        base = tid * _B_PER_T
        pltpu.sync_copy(idx_hbm.at[tid], idx_v)
        pltpu.async_copy(table_hbm.at[idx_v], rows_v, sem).wait()
        pltpu.sync_copy(rows_v, out_hbm.at[pl.ds(base, _B_PER_T)])

    return gather_kernel


def kernel(times, emb_weight):
    gather = _make_gather()
    idx = times.astype(jnp.int32).reshape(_NTILES, _B_PER_T)
    return gather(idx, emb_weight)


# speedup vs baseline: 1.0829x; 1.0000x over previous
import functools

import jax
import jax.numpy as jnp
from jax import lax
from jax.experimental import pallas as pl
from jax.experimental.pallas import tpu as pltpu
from jax.experimental.pallas import tpu_sc as plsc

_DIM = 128
_BATCH = 4096
_NTILES = 16
_B_PER_T = _BATCH // _NTILES


def _make_gather():
    mesh = plsc.VectorSubcoreMesh(
        core_axis_name="c", subcore_axis_name="s", num_cores=1
    )

    @functools.partial(
        pl.kernel,
        mesh=mesh,
        out_type=jax.ShapeDtypeStruct((_BATCH, _DIM), jnp.float32),
        compiler_params=pltpu.CompilerParams(
            disable_bounds_checks=True,
            disable_semaphore_checks=True,
            skip_device_barrier=True,
        ),
        scratch_types=[
            pltpu.VMEM((_B_PER_T,), jnp.int32),
            pltpu.VMEM((_B_PER_T, _DIM), jnp.float32),
            pltpu.SemaphoreType.DMA,
        ],
    )
    def gather_kernel(idx_hbm, table_hbm, out_hbm, idx_v, rows_v, sem):
        tid = lax.axis_index("s")
        base = tid * _B_PER_T
        pltpu.sync_copy(idx_hbm.at[tid], idx_v)
        pltpu.async_copy(table_hbm.at[idx_v], rows_v, sem).wait()
        pltpu.sync_copy(rows_v, out_hbm.at[pl.ds(base, _B_PER_T)])

    return gather_kernel


def kernel(times, emb_weight):
    gather = _make_gather()
    idx = times.astype(jnp.int32).reshape(_NTILES, _B_PER_T)
    return gather(idx, emb_weight)
